# Initial kernel scaffold; baseline (speedup 1.0000x reference)
import jax
import jax.numpy as jnp
from jax.experimental import pallas as pl


def kernel(x, edge_index, edge_attr, masked_atom_mask, enc_W1, enc_b1, enc_W2, enc_b2, prelu_a, W_n2d, dec_W, dec_b):
    raise NotImplementedError("write your pallas kernel here")


# speedup vs baseline: 1.0281x; 1.0281x over previous
import functools

import jax
import jax.numpy as jnp
from jax import lax
from jax.experimental import pallas as pl
from jax.experimental.pallas import tpu as pltpu
from jax.experimental.pallas import tpu_sc as plsc

N = 10000
E = 160000
D = 500
OUT = 119
L = 5

D2 = 512
TN = N + 16
TE = E + 16
NC = 2
NS = 16
NW = NC * NS
LANE = 16

TR = 160
ROUNDS = 2
NPAD = ROUNDS * NW * TR
EPW = E // NW
EPW_PAD = 5008
GROUPS = EPW_PAD // LANE
PB = 48
NPB = 5056
NOWN = ROUNDS * NW


def _bucket_body(src_hbm, dst_hbm, psrc_hbm, peid_hbm, counts_hbm,
                 src_v, dst_v, psrc_v, peid_v, cbuf):
    core = lax.axis_index("c")
    sub = lax.axis_index("s")
    wid = sub * NC + core
    ebase = wid * EPW
    pltpu.sync_copy(src_hbm.at[pl.ds(ebase, EPW)], src_v.at[pl.ds(0, EPW)])
    pltpu.sync_copy(dst_hbm.at[pl.ds(ebase, EPW)], dst_v.at[pl.ds(0, EPW)])
    ii = lax.iota(jnp.int32, LANE)
    plsc.store_scatter(dst_v, [EPW + ii], jnp.full((LANE,), -(2 ** 20), jnp.int32),
                       mask=ii < (EPW_PAD - EPW))

    def per_owner(o, carry):
        c0, c1, c2, c3 = carry
        obase = o * TR

        def scan(i, cnt):
            s16 = src_v[pl.ds(i * LANE, LANE)]
            d16 = dst_v[pl.ds(i * LANE, LANE)]
            dl = d16 - obase
            m = (dl >= 0) & (dl < TR)
            cs = plsc.cumsum(m.astype(jnp.int32))
            pos = cnt + cs - 1
            e16 = ebase + i * LANE + ii
            plsc.store_scatter(psrc_v, [pos], s16 * 256 + dl, mask=m)
            plsc.store_scatter(peid_v, [pos], e16 * 256 + dl, mask=m)
            return cnt + cs[15]

        cnt = lax.fori_loop(0, GROUPS, scan, jnp.int32(0))
        for j in range(PB // LANE):
            plsc.store_scatter(psrc_v, [cnt + j * LANE + ii],
                               jnp.full((LANE,), N * 256, jnp.int32))
            plsc.store_scatter(peid_v, [cnt + j * LANE + ii],
                               jnp.full((LANE,), E * 256, jnp.int32))
        region = (wid * NOWN + o) * NPB
        nb = (cnt + PB - 1) // PB

        def wr(j, _):
            pltpu.sync_copy(psrc_v.at[pl.ds(j * PB, PB)],
                            psrc_hbm.at[pl.ds(region + j * PB, PB)])
            pltpu.sync_copy(peid_v.at[pl.ds(j * PB, PB)],
                            peid_hbm.at[pl.ds(region + j * PB, PB)])
            return 0

        lax.fori_loop(0, nb, wr, 0)
        lane = o % LANE
        q = o // LANE
        c0 = jnp.where((q == 0) & (ii == lane), cnt, c0)
        c1 = jnp.where((q == 1) & (ii == lane), cnt, c1)
        c2 = jnp.where((q == 2) & (ii == lane), cnt, c2)
        c3 = jnp.where((q == 3) & (ii == lane), cnt, c3)
        return (c0, c1, c2, c3)

    z16 = jnp.zeros((LANE,), jnp.int32)
    c0, c1, c2, c3 = lax.fori_loop(0, NOWN, per_owner, (z16, z16, z16, z16))
    cbuf[pl.ds(0, LANE)] = c0
    cbuf[pl.ds(16, LANE)] = c1
    cbuf[pl.ds(32, LANE)] = c2
    cbuf[pl.ds(48, LANE)] = c3
    pltpu.sync_copy(cbuf, counts_hbm.at[wid])


_bucket = pl.kernel(
    _bucket_body,
    out_type=(
        jax.ShapeDtypeStruct((NW * NOWN * NPB,), jnp.int32),
        jax.ShapeDtypeStruct((NW * NOWN * NPB,), jnp.int32),
        jax.ShapeDtypeStruct((NW, NOWN), jnp.int32),
    ),
    mesh=plsc.VectorSubcoreMesh(core_axis_name="c", subcore_axis_name="s",
                                num_cores=NC, num_subcores=NS),
    compiler_params=pltpu.CompilerParams(needs_layout_passes=False),
    scratch_types=[
        pltpu.VMEM((EPW_PAD,), jnp.int32),
        pltpu.VMEM((EPW_PAD,), jnp.int32),
        pltpu.VMEM((NPB,), jnp.int32),
        pltpu.VMEM((NPB,), jnp.int32),
        pltpu.VMEM((NOWN,), jnp.int32),
    ],
)


def _pass_body(table, pair_hbm, countsT_hbm, zeros_hbm, out,
               crow, pair_v, gidx_b, rows_v, acc, sem):
    core = lax.axis_index("c")
    sub = lax.axis_index("s")
    wid = sub * NC + core
    ii = lax.iota(jnp.int32, LANE)
    for r in range(ROUNDS):
        o = r * NW + wid
        obase = o * TR
        pltpu.sync_copy(zeros_hbm, acc)
        pltpu.sync_copy(countsT_hbm.at[o], crow)
        cr0 = crow[pl.ds(0, LANE)]
        cr1 = crow[pl.ds(LANE, LANE)]

        def per_scanner(w2, _):
            wsel = jnp.where(w2 < LANE, cr0, cr1)
            wm_splat = jnp.zeros((LANE,), jnp.int32) + (w2 % LANE)
            cnt = wsel.at[wm_splat].get(mode="promise_in_bounds")[0]
            region = (w2 * NOWN + o) * NPB
            nb = (cnt + PB - 1) // PB

            def bat(j, _):
                pltpu.sync_copy(pair_hbm.at[pl.ds(region + j * PB, PB)], pair_v)
                dls = []
                for q in range(PB // LANE):
                    v = pair_v[pl.ds(q * LANE, LANE)]
                    gidx_b[pl.ds(q * LANE, LANE)] = v >> 8
                    dls.append(v & 255)
                pltpu.async_copy(table.at[gidx_b], rows_v, sem).wait()

                def acc_row(k, _):
                    kq = k // LANE
                    km_splat = jnp.zeros((LANE,), jnp.int32) + (k % LANE)
                    dsel = dls[-1]
                    for q in range(PB // LANE - 1):
                        dsel = jnp.where(kq == q, dls[q], dsel)
                    row_idx = dsel.at[km_splat].get(mode="promise_in_bounds")
                    krow = jnp.zeros((LANE,), jnp.int32) + k
                    for c in range(D2 // LANE):
                        col_idx = c * LANE + ii
                        vals = plsc.load_gather(rows_v, [krow, col_idx])
                        plsc.addupdate_scatter(acc, [row_idx, col_idx], vals)
                    return 0

                lax.fori_loop(0, PB, acc_row, 0)
                return 0

            lax.fori_loop(0, nb, bat, 0)
            return 0

        lax.fori_loop(0, NW, per_scanner, 0)
        pltpu.sync_copy(acc, out.at[pl.ds(obase, TR)])


def _make_pass():
    return pl.kernel(
        _pass_body,
        out_type=jax.ShapeDtypeStruct((NPAD, D2), jnp.float32),
        mesh=plsc.VectorSubcoreMesh(core_axis_name="c", subcore_axis_name="s",
                                    num_cores=NC, num_subcores=NS),
        compiler_params=pltpu.CompilerParams(needs_layout_passes=False),
        scratch_types=[
            pltpu.VMEM((NW,), jnp.int32),
            pltpu.VMEM((PB,), jnp.int32),
            pltpu.VMEM((PB,), jnp.int32),
            pltpu.VMEM((PB, D2), jnp.float32),
            pltpu.VMEM((TR, D2), jnp.float32),
            pltpu.SemaphoreType.DMA,
        ],
    )


_sc_pass = _make_pass()


BM = 2504
BMD = 1000


def _layer_body(relu_out, h_ref, c_ref, a_ref, w1_ref, b1_ref, w2_ref, b2_ref,
                o_ref):
    z = h_ref[...] + c_ref[...] + a_ref[...]
    z = jnp.dot(z, w1_ref[...], preferred_element_type=jnp.float32) + b1_ref[...]
    z = jnp.maximum(z, 0.0)
    z = jnp.dot(z, w2_ref[...], preferred_element_type=jnp.float32) + b2_ref[...]
    if relu_out:
        z = jnp.maximum(z, 0.0)
    rows = pl.program_id(0) * BM + lax.broadcasted_iota(jnp.int32, (BM, 1), 0)
    o_ref[...] = jnp.where(rows < N, z, 0.0)


def _make_layer(relu_out):
    return pl.pallas_call(
        functools.partial(_layer_body, relu_out),
        grid=(TN // BM,),
        in_specs=[
            pl.BlockSpec((BM, D2), lambda i: (i, 0)),
            pl.BlockSpec((BM, D2), lambda i: (i, 0)),
            pl.BlockSpec((BM, D2), lambda i: (i, 0)),
            pl.BlockSpec((D2, D2), lambda i: (0, 0)),
            pl.BlockSpec((1, D2), lambda i: (0, 0)),
            pl.BlockSpec((D2, D2), lambda i: (0, 0)),
            pl.BlockSpec((1, D2), lambda i: (0, 0)),
        ],
        out_specs=pl.BlockSpec((BM, D2), lambda i: (i, 0)),
        out_shape=jax.ShapeDtypeStruct((TN, D2), jnp.float32),
    )


def _n2d_body(h_ref, keep_ref, w_ref, a_ref, o_ref):
    h = h_ref[...]
    a = a_ref[0, 0]
    z = jnp.where(h >= 0.0, h, a * h)
    z = jnp.dot(z, w_ref[...], preferred_element_type=jnp.float32)
    o_ref[...] = z * keep_ref[...]


_n2d = pl.pallas_call(
    _n2d_body,
    grid=(TN // BM,),
    in_specs=[
        pl.BlockSpec((BM, D2), lambda i: (i, 0)),
        pl.BlockSpec((BM, 1), lambda i: (i, 0)),
        pl.BlockSpec((D2, D2), lambda i: (0, 0)),
        pl.BlockSpec((1, 1), lambda i: (0, 0)),
    ],
    out_specs=pl.BlockSpec((BM, D2), lambda i: (i, 0)),
    out_shape=jax.ShapeDtypeStruct((TN, D2), jnp.float32),
)


def _dec_body(h_ref, c_ref, a_ref, w_ref, b_ref, o_ref):
    z = h_ref[...] + c_ref[...] + a_ref[...]
    o_ref[...] = jnp.dot(z, w_ref[...], preferred_element_type=jnp.float32) + b_ref[...]


_dec = pl.pallas_call(
    _dec_body,
    grid=(N // BMD,),
    in_specs=[
        pl.BlockSpec((BMD, D2), lambda i: (i, 0)),
        pl.BlockSpec((BMD, D2), lambda i: (i, 0)),
        pl.BlockSpec((BMD, D2), lambda i: (i, 0)),
        pl.BlockSpec((D2, OUT), lambda i: (0, 0)),
        pl.BlockSpec((1, OUT), lambda i: (0, 0)),
    ],
    out_specs=pl.BlockSpec((BMD, OUT), lambda i: (i, 0)),
    out_shape=jax.ShapeDtypeStruct((N, OUT), jnp.float32),
)

_layer_mid = _make_layer(True)
_layer_last = _make_layer(False)


def kernel(x, edge_index, edge_attr, masked_atom_mask, enc_W1, enc_b1,
           enc_W2, enc_b2, prelu_a, W_n2d, dec_W, dec_b):
    src = edge_index[0]
    dst = edge_index[1]
    x_pad = jnp.pad(x, ((0, TN - N), (0, D2 - D)))
    ea_pad = jnp.pad(edge_attr, ((0, TE - E), (0, D2 - D)))
    w1p = jnp.pad(enc_W1, ((0, 0), (0, D2 - D), (0, D2 - D)))
    b1p = jnp.pad(enc_b1, ((0, 0), (0, D2 - D)))
    w2p = jnp.pad(enc_W2, ((0, 0), (0, D2 - D), (0, D2 - D)))
    b2p = jnp.pad(enc_b2, ((0, 0), (0, D2 - D)))
    wn2dp = jnp.pad(W_n2d, ((0, D2 - D), (0, D2 - D)))
    dec_wp = jnp.pad(dec_W, ((0, D2 - D), (0, 0)))
    keep = jnp.pad(1.0 - masked_atom_mask.astype(jnp.float32).reshape(N, 1),
                   ((0, TN - N), (0, 0)))
    zeros_stage = jnp.zeros((TR, D2), jnp.float32)

    psrc, peid, counts = _bucket(src, dst)
    counts_t = counts.T

    C = _sc_pass(ea_pad, peid, counts_t, zeros_stage)

    h = x_pad
    for l in range(L):
        agg = _sc_pass(h, psrc, counts_t, zeros_stage)
        layer = _layer_mid if l < L - 1 else _layer_last
        h = layer(h, C, agg, w1p[l], b1p[l].reshape(1, D2),
                  w2p[l], b2p[l].reshape(1, D2))

    h = _n2d(h, keep, wn2dp, prelu_a.reshape(1, 1))

    agg = _sc_pass(h, psrc, counts_t, zeros_stage)
    return _dec(h, C, agg, dec_wp, dec_b.reshape(1, OUT))


# speedup vs baseline: 1.8807x; 1.8293x over previous
import functools

import jax
import jax.numpy as jnp
from jax import lax
from jax.experimental import pallas as pl
from jax.experimental.pallas import tpu as pltpu
from jax.experimental.pallas import tpu_sc as plsc

N = 10000
E = 160000
D = 500
OUT = 119
L = 5

D2 = 512
TN = N + 16
TE = E + 16
NC = 2
NS = 16
NW = NC * NS
LANE = 16

TR = 160
ROUNDS = 2
NPAD = ROUNDS * NW * TR
NOWN = ROUNDS * NW
PB = 32
CE = 10000
NCHUNK = E // CE
GROUPS = CE // LANE
FB = 960
BUFCAP = 1024
CAP = PB + E + PB // 1


def _flush_ready(buf_s, buf_e, strip_s, strip_e, sbase, cnt, doff):
    def do_flush(args):
        c, d = args
        off = pl.multiple_of(sbase + d, 8)
        pltpu.sync_copy(buf_s.at[pl.ds(0, FB)], strip_s.at[pl.ds(off, FB)])
        pltpu.sync_copy(buf_e.at[pl.ds(0, FB)], strip_e.at[pl.ds(off, FB)])
        ii = lax.iota(jnp.int32, LANE)
        resid = c - FB
        vs = buf_s[pl.ds(FB, LANE)]
        ve = buf_e[pl.ds(FB, LANE)]
        plsc.store_scatter(buf_s, [ii], vs, mask=ii < resid)
        plsc.store_scatter(buf_e, [ii], ve, mask=ii < resid)
        return (resid, d + FB)

    return lax.cond(cnt >= FB, do_flush, lambda a: a, (cnt, doff))


def _strips_body(src_hbm, dst_hbm, ssrc_hbm, seid_hbm,
                 src_v, dst_v, bs0, be0, bs1, be1):
    core = lax.axis_index("c")
    sub = lax.axis_index("s")
    wid = sub * NC + core
    ii = lax.iota(jnp.int32, LANE)
    o0 = wid
    o1 = NW + wid
    ob0 = o0 * TR
    ob1 = o1 * TR
    sb0 = o0 * CAP
    sb1 = o1 * CAP

    def per_chunk(ch, carry):
        c0, d0, c1, d1 = carry
        pltpu.sync_copy(src_hbm.at[pl.ds(ch * CE, CE)], src_v)
        pltpu.sync_copy(dst_hbm.at[pl.ds(ch * CE, CE)], dst_v)

        def per_group(i, gc):
            g0, gd0, g1, gd1 = gc
            s16 = src_v[pl.ds(i * LANE, LANE)]
            d16 = dst_v[pl.ds(i * LANE, LANE)]
            e16 = ch * CE + i * LANE + ii
            dl0 = d16 - ob0
            m0 = (dl0 >= 0) & (dl0 < TR)
            cs0 = plsc.cumsum(m0.astype(jnp.int32))
            pos0 = g0 + cs0 - 1
            plsc.store_scatter(bs0, [pos0], s16 * 256 + dl0, mask=m0)
            plsc.store_scatter(be0, [pos0], e16 * 256 + dl0, mask=m0)
            g0 = g0 + cs0[15]
            dl1 = d16 - ob1
            m1 = (dl1 >= 0) & (dl1 < TR)
            cs1 = plsc.cumsum(m1.astype(jnp.int32))
            pos1 = g1 + cs1 - 1
            plsc.store_scatter(bs1, [pos1], s16 * 256 + dl1, mask=m1)
            plsc.store_scatter(be1, [pos1], e16 * 256 + dl1, mask=m1)
            g1 = g1 + cs1[15]
            g0, gd0 = _flush_ready(bs0, be0, ssrc_hbm, seid_hbm, sb0 + PB, g0, gd0)
            g1, gd1 = _flush_ready(bs1, be1, ssrc_hbm, seid_hbm, sb1 + PB, g1, gd1)
            return (g0, gd0, g1, gd1)

        return lax.fori_loop(0, GROUPS, per_group, (c0, d0, c1, d1))

    z = jnp.int32(0)
    c0, d0, c1, d1 = lax.fori_loop(0, NCHUNK, per_chunk, (z, z, z, z))

    def finish(buf_s, buf_e, sbase, cnt, doff):
        for j in range(PB // LANE):
            plsc.store_scatter(buf_s, [cnt + j * LANE + ii],
                               jnp.full((LANE,), N * 256, jnp.int32))
            plsc.store_scatter(buf_e, [cnt + j * LANE + ii],
                               jnp.full((LANE,), E * 256, jnp.int32))
        nb = (cnt + PB - 1) // PB

        def wr(j, _):
            off = pl.multiple_of(sbase + PB + doff + j * PB, 8)
            pltpu.sync_copy(buf_s.at[pl.ds(j * PB, PB)],
                            ssrc_hbm.at[pl.ds(off, PB)])
            pltpu.sync_copy(buf_e.at[pl.ds(j * PB, PB)],
                            seid_hbm.at[pl.ds(off, PB)])
            return 0

        lax.fori_loop(0, nb, wr, 0)
        slen = doff + nb * PB
        hdr = jnp.where(ii == 0, slen * 256, N * 256)
        plsc.store_scatter(buf_s, [ii], hdr)
        plsc.store_scatter(buf_e, [ii], jnp.where(ii == 0, slen * 256, E * 256))
        for j in range(1, PB // LANE):
            plsc.store_scatter(buf_s, [j * LANE + ii],
                               jnp.full((LANE,), N * 256, jnp.int32))
            plsc.store_scatter(buf_e, [j * LANE + ii],
                               jnp.full((LANE,), E * 256, jnp.int32))
        pltpu.sync_copy(buf_s.at[pl.ds(0, PB)], ssrc_hbm.at[pl.ds(sbase, PB)])
        pltpu.sync_copy(buf_e.at[pl.ds(0, PB)], seid_hbm.at[pl.ds(sbase, PB)])

    finish(bs0, be0, sb0, c0, d0)
    finish(bs1, be1, sb1, c1, d1)


_strips = pl.kernel(
    _strips_body,
    out_type=(
        jax.ShapeDtypeStruct((NOWN * CAP,), jnp.int32),
        jax.ShapeDtypeStruct((NOWN * CAP,), jnp.int32),
    ),
    mesh=plsc.VectorSubcoreMesh(core_axis_name="c", subcore_axis_name="s",
                                num_cores=NC, num_subcores=NS),
    compiler_params=pltpu.CompilerParams(needs_layout_passes=False),
    scratch_types=[
        pltpu.VMEM((CE,), jnp.int32),
        pltpu.VMEM((CE,), jnp.int32),
        pltpu.VMEM((BUFCAP,), jnp.int32),
        pltpu.VMEM((BUFCAP,), jnp.int32),
        pltpu.VMEM((BUFCAP,), jnp.int32),
        pltpu.VMEM((BUFCAP,), jnp.int32),
    ],
)


def _pass_body(table, strip_hbm, zeros_hbm, out,
               pair_a, pair_b, gidx_a, gidx_b, dl_a, dl_b,
               rows_a, rows_b, acc, sem_a, sem_b):
    core = lax.axis_index("c")
    sub = lax.axis_index("s")
    wid = sub * NC + core
    ii = lax.iota(jnp.int32, LANE)

    def fetch(j, sbase, pair_v, gidx_v, dl_v):
        pltpu.sync_copy(strip_hbm.at[pl.ds(sbase + PB + j * PB, PB)], pair_v)
        for q in range(PB // LANE):
            v = pair_v[pl.ds(q * LANE, LANE)]
            gidx_v[pl.ds(q * LANE, LANE)] = v >> 8
            dl_v[pl.ds(q * LANE, LANE)] = v & 255

    def accrow(dl_v, rows_v):
        d0 = dl_v[pl.ds(0, LANE)]
        d1 = dl_v[pl.ds(LANE, LANE)]

        def body(k, _):
            km_splat = jnp.zeros((LANE,), jnp.int32) + (k % LANE)
            dsel = jnp.where(k < LANE, d0, d1)
            row_idx = dsel.at[km_splat].get(mode="promise_in_bounds")
            krow = jnp.zeros((LANE,), jnp.int32) + k
            for c in range(D2 // LANE):
                col_idx = c * LANE + ii
                vals = plsc.load_gather(rows_v, [krow, col_idx])
                plsc.addupdate_scatter(acc, [row_idx, col_idx], vals)
            return 0

        lax.fori_loop(0, PB, body, 0)

    for r in range(ROUNDS):
        o = r * NW + wid
        obase = o * TR
        sbase = o * CAP
        pltpu.sync_copy(zeros_hbm, acc)
        pltpu.sync_copy(strip_hbm.at[pl.ds(sbase, PB)], pair_a)
        hv = pair_a[pl.ds(0, LANE)]
        nb = (hv[0] >> 8) // PB

        @pl.when(nb > 0)
        def _prime():
            fetch(0, sbase, pair_a, gidx_a, dl_a)
            pltpu.async_copy(table.at[gidx_a], rows_a, sem_a)

        def body2(i, _):
            j1 = 2 * i + 1

            @pl.when(j1 < nb)
            def _fire_b():
                fetch(j1, sbase, pair_b, gidx_b, dl_b)
                pltpu.async_copy(table.at[gidx_b], rows_b, sem_b)

            pltpu.make_async_copy(table.at[gidx_a], rows_a, sem_a).wait()
            accrow(dl_a, rows_a)

            @pl.when(j1 + 1 < nb)
            def _fire_a():
                fetch(j1 + 1, sbase, pair_a, gidx_a, dl_a)
                pltpu.async_copy(table.at[gidx_a], rows_a, sem_a)

            @pl.when(j1 < nb)
            def _drain_b():
                pltpu.make_async_copy(table.at[gidx_b], rows_b, sem_b).wait()
                accrow(dl_b, rows_b)

            return 0

        lax.fori_loop(0, (nb + 1) // 2, body2, 0)
        pltpu.sync_copy(acc, out.at[pl.ds(obase, TR)])


def _make_pass():
    return pl.kernel(
        _pass_body,
        out_type=jax.ShapeDtypeStruct((NPAD, D2), jnp.float32),
        mesh=plsc.VectorSubcoreMesh(core_axis_name="c", subcore_axis_name="s",
                                    num_cores=NC, num_subcores=NS),
        compiler_params=pltpu.CompilerParams(needs_layout_passes=False),
        scratch_types=[
            pltpu.VMEM((PB,), jnp.int32),
            pltpu.VMEM((PB,), jnp.int32),
            pltpu.VMEM((PB,), jnp.int32),
            pltpu.VMEM((PB,), jnp.int32),
            pltpu.VMEM((PB,), jnp.int32),
            pltpu.VMEM((PB,), jnp.int32),
            pltpu.VMEM((PB, D2), jnp.float32),
            pltpu.VMEM((PB, D2), jnp.float32),
            pltpu.VMEM((TR, D2), jnp.float32),
            pltpu.SemaphoreType.DMA,
            pltpu.SemaphoreType.DMA,
        ],
    )


_sc_pass = _make_pass()


BM = 2504
BMD = 1000


def _layer_body(relu_out, h_ref, c_ref, a_ref, w1_ref, b1_ref, w2_ref, b2_ref,
                o_ref):
    z = h_ref[...] + c_ref[...] + a_ref[...]
    z = jnp.dot(z, w1_ref[...], preferred_element_type=jnp.float32) + b1_ref[...]
    z = jnp.maximum(z, 0.0)
    z = jnp.dot(z, w2_ref[...], preferred_element_type=jnp.float32) + b2_ref[...]
    if relu_out:
        z = jnp.maximum(z, 0.0)
    rows = pl.program_id(0) * BM + lax.broadcasted_iota(jnp.int32, (BM, 1), 0)
    o_ref[...] = jnp.where(rows < N, z, 0.0)


def _make_layer(relu_out):
    return pl.pallas_call(
        functools.partial(_layer_body, relu_out),
        grid=(TN // BM,),
        in_specs=[
            pl.BlockSpec((BM, D2), lambda i: (i, 0)),
            pl.BlockSpec((BM, D2), lambda i: (i, 0)),
            pl.BlockSpec((BM, D2), lambda i: (i, 0)),
            pl.BlockSpec((D2, D2), lambda i: (0, 0)),
            pl.BlockSpec((1, D2), lambda i: (0, 0)),
            pl.BlockSpec((D2, D2), lambda i: (0, 0)),
            pl.BlockSpec((1, D2), lambda i: (0, 0)),
        ],
        out_specs=pl.BlockSpec((BM, D2), lambda i: (i, 0)),
        out_shape=jax.ShapeDtypeStruct((TN, D2), jnp.float32),
    )


def _n2d_body(h_ref, keep_ref, w_ref, a_ref, o_ref):
    h = h_ref[...]
    a = a_ref[0, 0]
    z = jnp.where(h >= 0.0, h, a * h)
    z = jnp.dot(z, w_ref[...], preferred_element_type=jnp.float32)
    o_ref[...] = z * keep_ref[...]


_n2d = pl.pallas_call(
    _n2d_body,
    grid=(TN // BM,),
    in_specs=[
        pl.BlockSpec((BM, D2), lambda i: (i, 0)),
        pl.BlockSpec((BM, 1), lambda i: (i, 0)),
        pl.BlockSpec((D2, D2), lambda i: (0, 0)),
        pl.BlockSpec((1, 1), lambda i: (0, 0)),
    ],
    out_specs=pl.BlockSpec((BM, D2), lambda i: (i, 0)),
    out_shape=jax.ShapeDtypeStruct((TN, D2), jnp.float32),
)


def _dec_body(h_ref, c_ref, a_ref, w_ref, b_ref, o_ref):
    z = h_ref[...] + c_ref[...] + a_ref[...]
    o_ref[...] = jnp.dot(z, w_ref[...], preferred_element_type=jnp.float32) + b_ref[...]


_dec = pl.pallas_call(
    _dec_body,
    grid=(N // BMD,),
    in_specs=[
        pl.BlockSpec((BMD, D2), lambda i: (i, 0)),
        pl.BlockSpec((BMD, D2), lambda i: (i, 0)),
        pl.BlockSpec((BMD, D2), lambda i: (i, 0)),
        pl.BlockSpec((D2, OUT), lambda i: (0, 0)),
        pl.BlockSpec((1, OUT), lambda i: (0, 0)),
    ],
    out_specs=pl.BlockSpec((BMD, OUT), lambda i: (i, 0)),
    out_shape=jax.ShapeDtypeStruct((N, OUT), jnp.float32),
)

_layer_mid = _make_layer(True)
_layer_last = _make_layer(False)


def kernel(x, edge_index, edge_attr, masked_atom_mask, enc_W1, enc_b1,
           enc_W2, enc_b2, prelu_a, W_n2d, dec_W, dec_b):
    src = edge_index[0]
    dst = edge_index[1]
    x_pad = jnp.pad(x, ((0, TN - N), (0, D2 - D)))
    ea_pad = jnp.pad(edge_attr, ((0, TE - E), (0, D2 - D)))
    w1p = jnp.pad(enc_W1, ((0, 0), (0, D2 - D), (0, D2 - D)))
    b1p = jnp.pad(enc_b1, ((0, 0), (0, D2 - D)))
    w2p = jnp.pad(enc_W2, ((0, 0), (0, D2 - D), (0, D2 - D)))
    b2p = jnp.pad(enc_b2, ((0, 0), (0, D2 - D)))
    wn2dp = jnp.pad(W_n2d, ((0, D2 - D), (0, D2 - D)))
    dec_wp = jnp.pad(dec_W, ((0, D2 - D), (0, 0)))
    keep = jnp.pad(1.0 - masked_atom_mask.astype(jnp.float32).reshape(N, 1),
                   ((0, TN - N), (0, 0)))
    zeros_stage = jnp.zeros((TR, D2), jnp.float32)

    ssrc, seid = _strips(src, dst)

    C = _sc_pass(ea_pad, seid, zeros_stage)

    h = x_pad
    for l in range(L):
        agg = _sc_pass(h, ssrc, zeros_stage)
        layer = _layer_mid if l < L - 1 else _layer_last
        h = layer(h, C, agg, w1p[l], b1p[l].reshape(1, D2),
                  w2p[l], b2p[l].reshape(1, D2))

    h = _n2d(h, keep, wn2dp, prelu_a.reshape(1, 1))

    agg = _sc_pass(h, ssrc, zeros_stage)
    return _dec(h, C, agg, dec_wp, dec_b.reshape(1, OUT))


# speedup vs baseline: 2.0034x; 1.0653x over previous
import functools

import jax
import jax.numpy as jnp
from jax import lax
from jax.experimental import pallas as pl
from jax.experimental.pallas import tpu as pltpu
from jax.experimental.pallas import tpu_sc as plsc

N = 10000
E = 160000
D = 500
OUT = 119
L = 5

D2 = 512
TN = N + 16
TE = E + 16
NC = 2
NS = 16
NW = NC * NS
LANE = 16

TR = 160
ROUNDS = 2
NPAD = ROUNDS * NW * TR
NOWN = ROUNDS * NW
PB = 32
CE = 10000
NCHUNK = E // CE
GROUPS = CE // LANE
FB = 960
BUFCAP = 1024
CAP = PB + E + PB // 1
PCAP = 6144
NBC = PCAP // PB


def _flush_ready(buf_s, buf_e, strip_s, strip_e, sbase, cnt, doff):
    def do_flush(args):
        c, d = args
        off = pl.multiple_of(sbase + d, 8)
        pltpu.sync_copy(buf_s.at[pl.ds(0, FB)], strip_s.at[pl.ds(off, FB)])
        pltpu.sync_copy(buf_e.at[pl.ds(0, FB)], strip_e.at[pl.ds(off, FB)])
        ii = lax.iota(jnp.int32, LANE)
        resid = c - FB
        vs = buf_s[pl.ds(FB, LANE)]
        ve = buf_e[pl.ds(FB, LANE)]
        plsc.store_scatter(buf_s, [ii], vs, mask=ii < resid)
        plsc.store_scatter(buf_e, [ii], ve, mask=ii < resid)
        return (resid, d + FB)

    return lax.cond(cnt >= FB, do_flush, lambda a: a, (cnt, doff))


def _strips_body(src_hbm, dst_hbm, ssrc_hbm, seid_hbm,
                 src_v, dst_v, bs0, be0, bs1, be1):
    core = lax.axis_index("c")
    sub = lax.axis_index("s")
    wid = sub * NC + core
    ii = lax.iota(jnp.int32, LANE)
    o0 = wid
    o1 = NW + wid
    ob0 = o0 * TR
    ob1 = o1 * TR
    sb0 = o0 * CAP
    sb1 = o1 * CAP

    def per_chunk(ch, carry):
        c0, d0, c1, d1 = carry
        pltpu.sync_copy(src_hbm.at[pl.ds(ch * CE, CE)], src_v)
        pltpu.sync_copy(dst_hbm.at[pl.ds(ch * CE, CE)], dst_v)

        def per_group(i, gc):
            g0, gd0, g1, gd1 = gc
            s16 = src_v[pl.ds(i * LANE, LANE)]
            d16 = dst_v[pl.ds(i * LANE, LANE)]
            e16 = ch * CE + i * LANE + ii
            dl0 = d16 - ob0
            m0 = (dl0 >= 0) & (dl0 < TR)
            cs0 = plsc.cumsum(m0.astype(jnp.int32))
            pos0 = g0 + cs0 - 1
            plsc.store_scatter(bs0, [pos0], s16 * 256 + dl0, mask=m0)
            plsc.store_scatter(be0, [pos0], e16 * 256 + dl0, mask=m0)
            g0 = g0 + cs0[15]
            dl1 = d16 - ob1
            m1 = (dl1 >= 0) & (dl1 < TR)
            cs1 = plsc.cumsum(m1.astype(jnp.int32))
            pos1 = g1 + cs1 - 1
            plsc.store_scatter(bs1, [pos1], s16 * 256 + dl1, mask=m1)
            plsc.store_scatter(be1, [pos1], e16 * 256 + dl1, mask=m1)
            g1 = g1 + cs1[15]
            g0, gd0 = _flush_ready(bs0, be0, ssrc_hbm, seid_hbm, sb0 + PB, g0, gd0)
            g1, gd1 = _flush_ready(bs1, be1, ssrc_hbm, seid_hbm, sb1 + PB, g1, gd1)
            return (g0, gd0, g1, gd1)

        return lax.fori_loop(0, GROUPS, per_group, (c0, d0, c1, d1))

    z = jnp.int32(0)
    c0, d0, c1, d1 = lax.fori_loop(0, NCHUNK, per_chunk, (z, z, z, z))

    def finish(buf_s, buf_e, sbase, cnt, doff):
        for j in range(PB // LANE):
            plsc.store_scatter(buf_s, [cnt + j * LANE + ii],
                               jnp.full((LANE,), N * 256, jnp.int32))
            plsc.store_scatter(buf_e, [cnt + j * LANE + ii],
                               jnp.full((LANE,), E * 256, jnp.int32))
        nb = (cnt + PB - 1) // PB

        def wr(j, _):
            off = pl.multiple_of(sbase + PB + doff + j * PB, 8)
            pltpu.sync_copy(buf_s.at[pl.ds(j * PB, PB)],
                            ssrc_hbm.at[pl.ds(off, PB)])
            pltpu.sync_copy(buf_e.at[pl.ds(j * PB, PB)],
                            seid_hbm.at[pl.ds(off, PB)])
            return 0

        lax.fori_loop(0, nb, wr, 0)
        slen = doff + nb * PB
        hdr = jnp.where(ii == 0, slen * 256, N * 256)
        plsc.store_scatter(buf_s, [ii], hdr)
        plsc.store_scatter(buf_e, [ii], jnp.where(ii == 0, slen * 256, E * 256))
        for j in range(1, PB // LANE):
            plsc.store_scatter(buf_s, [j * LANE + ii],
                               jnp.full((LANE,), N * 256, jnp.int32))
            plsc.store_scatter(buf_e, [j * LANE + ii],
                               jnp.full((LANE,), E * 256, jnp.int32))
        pltpu.sync_copy(buf_s.at[pl.ds(0, PB)], ssrc_hbm.at[pl.ds(sbase, PB)])
        pltpu.sync_copy(buf_e.at[pl.ds(0, PB)], seid_hbm.at[pl.ds(sbase, PB)])

    finish(bs0, be0, sb0, c0, d0)
    finish(bs1, be1, sb1, c1, d1)


_strips = pl.kernel(
    _strips_body,
    out_type=(
        jax.ShapeDtypeStruct((NOWN * CAP + PCAP,), jnp.int32),
        jax.ShapeDtypeStruct((NOWN * CAP + PCAP,), jnp.int32),
    ),
    mesh=plsc.VectorSubcoreMesh(core_axis_name="c", subcore_axis_name="s",
                                num_cores=NC, num_subcores=NS),
    compiler_params=pltpu.CompilerParams(needs_layout_passes=False),
    scratch_types=[
        pltpu.VMEM((CE,), jnp.int32),
        pltpu.VMEM((CE,), jnp.int32),
        pltpu.VMEM((BUFCAP,), jnp.int32),
        pltpu.VMEM((BUFCAP,), jnp.int32),
        pltpu.VMEM((BUFCAP,), jnp.int32),
        pltpu.VMEM((BUFCAP,), jnp.int32),
    ],
)


def _pass_body(table, strip_hbm, zeros_hbm, out,
               pair_ch, hdr_v, gidx_a, gidx_b, dl_a, dl_b,
               rows_a, rows_b, acc, sem_a, sem_b):
    core = lax.axis_index("c")
    sub = lax.axis_index("s")
    wid = sub * NC + core
    ii = lax.iota(jnp.int32, LANE)

    def fetch(j, sbase, gidx_v, dl_v):
        @pl.when(j % NBC == 0)
        def _refill():
            off = pl.multiple_of(sbase + PB + j * PB, 8)
            pltpu.sync_copy(strip_hbm.at[pl.ds(off, PCAP)], pair_ch)

        local = pl.multiple_of((j % NBC) * PB, 8)
        for q in range(PB // LANE):
            v = pair_ch[pl.ds(local + q * LANE, LANE)]
            gidx_v[pl.ds(q * LANE, LANE)] = v >> 8
            dl_v[pl.ds(q * LANE, LANE)] = v & 255

    def accrow(dl_v, rows_v):
        d0 = dl_v[pl.ds(0, LANE)]
        d1 = dl_v[pl.ds(LANE, LANE)]

        def body(k, _):
            km_splat = jnp.zeros((LANE,), jnp.int32) + (k % LANE)
            dsel = jnp.where(k < LANE, d0, d1)
            row_idx = dsel.at[km_splat].get(mode="promise_in_bounds")
            krow = jnp.zeros((LANE,), jnp.int32) + k
            for c in range(D2 // LANE):
                col_idx = c * LANE + ii
                vals = plsc.load_gather(rows_v, [krow, col_idx])
                plsc.addupdate_scatter(acc, [row_idx, col_idx], vals)
            return 0

        lax.fori_loop(0, PB, body, 0)

    for r in range(ROUNDS):
        o = r * NW + wid
        obase = o * TR
        sbase = o * CAP
        pltpu.sync_copy(zeros_hbm, acc)
        pltpu.sync_copy(strip_hbm.at[pl.ds(sbase, PB)], hdr_v)
        hv = hdr_v[pl.ds(0, LANE)]
        nb = (hv[0] >> 8) // PB

        @pl.when(nb > 0)
        def _prime():
            fetch(0, sbase, gidx_a, dl_a)
            pltpu.async_copy(table.at[gidx_a], rows_a, sem_a)

        def body2(i, _):
            j1 = 2 * i + 1

            @pl.when(j1 < nb)
            def _fire_b():
                fetch(j1, sbase, gidx_b, dl_b)
                pltpu.async_copy(table.at[gidx_b], rows_b, sem_b)

            pltpu.make_async_copy(table.at[gidx_a], rows_a, sem_a).wait()
            accrow(dl_a, rows_a)

            @pl.when(j1 + 1 < nb)
            def _fire_a():
                fetch(j1 + 1, sbase, gidx_a, dl_a)
                pltpu.async_copy(table.at[gidx_a], rows_a, sem_a)

            @pl.when(j1 < nb)
            def _drain_b():
                pltpu.make_async_copy(table.at[gidx_b], rows_b, sem_b).wait()
                accrow(dl_b, rows_b)

            return 0

        lax.fori_loop(0, (nb + 1) // 2, body2, 0)
        pltpu.sync_copy(acc, out.at[pl.ds(obase, TR)])


def _make_pass():
    return pl.kernel(
        _pass_body,
        out_type=jax.ShapeDtypeStruct((NPAD, D2), jnp.float32),
        mesh=plsc.VectorSubcoreMesh(core_axis_name="c", subcore_axis_name="s",
                                    num_cores=NC, num_subcores=NS),
        compiler_params=pltpu.CompilerParams(needs_layout_passes=False),
        scratch_types=[
            pltpu.VMEM((PCAP,), jnp.int32),
            pltpu.VMEM((PB,), jnp.int32),
            pltpu.VMEM((PB,), jnp.int32),
            pltpu.VMEM((PB,), jnp.int32),
            pltpu.VMEM((PB,), jnp.int32),
            pltpu.VMEM((PB,), jnp.int32),
            pltpu.VMEM((PB, D2), jnp.float32),
            pltpu.VMEM((PB, D2), jnp.float32),
            pltpu.VMEM((TR, D2), jnp.float32),
            pltpu.SemaphoreType.DMA,
            pltpu.SemaphoreType.DMA,
        ],
    )


_sc_pass = _make_pass()


BM = 2504
BMD = 1000


def _layer_body(relu_out, h_ref, c_ref, a_ref, w1_ref, b1_ref, w2_ref, b2_ref,
                o_ref):
    z = h_ref[...] + c_ref[...] + a_ref[...]
    z = jnp.dot(z, w1_ref[...], preferred_element_type=jnp.float32) + b1_ref[...]
    z = jnp.maximum(z, 0.0)
    z = jnp.dot(z, w2_ref[...], preferred_element_type=jnp.float32) + b2_ref[...]
    if relu_out:
        z = jnp.maximum(z, 0.0)
    rows = pl.program_id(0) * BM + lax.broadcasted_iota(jnp.int32, (BM, 1), 0)
    o_ref[...] = jnp.where(rows < N, z, 0.0)


def _make_layer(relu_out):
    return pl.pallas_call(
        functools.partial(_layer_body, relu_out),
        grid=(TN // BM,),
        in_specs=[
            pl.BlockSpec((BM, D2), lambda i: (i, 0)),
            pl.BlockSpec((BM, D2), lambda i: (i, 0)),
            pl.BlockSpec((BM, D2), lambda i: (i, 0)),
            pl.BlockSpec((D2, D2), lambda i: (0, 0)),
            pl.BlockSpec((1, D2), lambda i: (0, 0)),
            pl.BlockSpec((D2, D2), lambda i: (0, 0)),
            pl.BlockSpec((1, D2), lambda i: (0, 0)),
        ],
        out_specs=pl.BlockSpec((BM, D2), lambda i: (i, 0)),
        out_shape=jax.ShapeDtypeStruct((TN, D2), jnp.float32),
    )


def _n2d_body(h_ref, keep_ref, w_ref, a_ref, o_ref):
    h = h_ref[...]
    a = a_ref[0, 0]
    z = jnp.where(h >= 0.0, h, a * h)
    z = jnp.dot(z, w_ref[...], preferred_element_type=jnp.float32)
    o_ref[...] = z * keep_ref[...]


_n2d = pl.pallas_call(
    _n2d_body,
    grid=(TN // BM,),
    in_specs=[
        pl.BlockSpec((BM, D2), lambda i: (i, 0)),
        pl.BlockSpec((BM, 1), lambda i: (i, 0)),
        pl.BlockSpec((D2, D2), lambda i: (0, 0)),
        pl.BlockSpec((1, 1), lambda i: (0, 0)),
    ],
    out_specs=pl.BlockSpec((BM, D2), lambda i: (i, 0)),
    out_shape=jax.ShapeDtypeStruct((TN, D2), jnp.float32),
)


def _dec_body(h_ref, c_ref, a_ref, w_ref, b_ref, o_ref):
    z = h_ref[...] + c_ref[...] + a_ref[...]
    o_ref[...] = jnp.dot(z, w_ref[...], preferred_element_type=jnp.float32) + b_ref[...]


_dec = pl.pallas_call(
    _dec_body,
    grid=(N // BMD,),
    in_specs=[
        pl.BlockSpec((BMD, D2), lambda i: (i, 0)),
        pl.BlockSpec((BMD, D2), lambda i: (i, 0)),
        pl.BlockSpec((BMD, D2), lambda i: (i, 0)),
        pl.BlockSpec((D2, OUT), lambda i: (0, 0)),
        pl.BlockSpec((1, OUT), lambda i: (0, 0)),
    ],
    out_specs=pl.BlockSpec((BMD, OUT), lambda i: (i, 0)),
    out_shape=jax.ShapeDtypeStruct((N, OUT), jnp.float32),
)

_layer_mid = _make_layer(True)
_layer_last = _make_layer(False)


def kernel(x, edge_index, edge_attr, masked_atom_mask, enc_W1, enc_b1,
           enc_W2, enc_b2, prelu_a, W_n2d, dec_W, dec_b):
    src = edge_index[0]
    dst = edge_index[1]
    x_pad = jnp.pad(x, ((0, TN - N), (0, D2 - D)))
    ea_pad = jnp.pad(edge_attr, ((0, TE - E), (0, D2 - D)))
    w1p = jnp.pad(enc_W1, ((0, 0), (0, D2 - D), (0, D2 - D)))
    b1p = jnp.pad(enc_b1, ((0, 0), (0, D2 - D)))
    w2p = jnp.pad(enc_W2, ((0, 0), (0, D2 - D), (0, D2 - D)))
    b2p = jnp.pad(enc_b2, ((0, 0), (0, D2 - D)))
    wn2dp = jnp.pad(W_n2d, ((0, D2 - D), (0, D2 - D)))
    dec_wp = jnp.pad(dec_W, ((0, D2 - D), (0, 0)))
    keep = jnp.pad(1.0 - masked_atom_mask.astype(jnp.float32).reshape(N, 1),
                   ((0, TN - N), (0, 0)))
    zeros_stage = jnp.zeros((TR, D2), jnp.float32)

    ssrc, seid = _strips(src, dst)

    C = _sc_pass(ea_pad, seid, zeros_stage)

    h = x_pad
    for l in range(L):
        agg = _sc_pass(h, ssrc, zeros_stage)
        layer = _layer_mid if l < L - 1 else _layer_last
        h = layer(h, C, agg, w1p[l], b1p[l].reshape(1, D2),
                  w2p[l], b2p[l].reshape(1, D2))

    h = _n2d(h, keep, wn2dp, prelu_a.reshape(1, 1))

    agg = _sc_pass(h, ssrc, zeros_stage)
    return _dec(h, C, agg, dec_wp, dec_b.reshape(1, OUT))


# speedup vs baseline: 2.1835x; 1.0899x over previous
import functools

import jax
import jax.numpy as jnp
from jax import lax
from jax.experimental import pallas as pl
from jax.experimental.pallas import tpu as pltpu
from jax.experimental.pallas import tpu_sc as plsc

N = 10000
E = 160000
D = 500
OUT = 119
L = 5

D2 = 512
TN = N + 16
TE = E + 16
NC = 2
NS = 16
NW = NC * NS
LANE = 16

TR = 160
ROUNDS = 2
NPAD = ROUNDS * NW * TR
NOWN = ROUNDS * NW
PB = 32
CE = 10000
NCHUNK = E // CE
GROUPS = CE // LANE
FB = 960
BUFCAP = 1024
CAP = PB + E + PB // 1
PCAP = 6144
NBC = PCAP // PB


def _flush_ready(buf_s, buf_e, strip_s, strip_e, sbase, cnt, doff):
    def do_flush(args):
        c, d = args
        off = pl.multiple_of(sbase + d, 8)
        pltpu.sync_copy(buf_s.at[pl.ds(0, FB)], strip_s.at[pl.ds(off, FB)])
        pltpu.sync_copy(buf_e.at[pl.ds(0, FB)], strip_e.at[pl.ds(off, FB)])
        ii = lax.iota(jnp.int32, LANE)
        resid = c - FB
        vs = buf_s[pl.ds(FB, LANE)]
        ve = buf_e[pl.ds(FB, LANE)]
        plsc.store_scatter(buf_s, [ii], vs, mask=ii < resid)
        plsc.store_scatter(buf_e, [ii], ve, mask=ii < resid)
        return (resid, d + FB)

    return lax.cond(cnt >= FB, do_flush, lambda a: a, (cnt, doff))


def _strips_body(src_hbm, dst_hbm, ssrc_hbm, seid_hbm,
                 src_v, dst_v, bs0, be0, bs1, be1):
    core = lax.axis_index("c")
    sub = lax.axis_index("s")
    wid = sub * NC + core
    ii = lax.iota(jnp.int32, LANE)
    o0 = wid
    o1 = NW + wid
    ob0 = o0 * TR
    ob1 = o1 * TR
    sb0 = o0 * CAP
    sb1 = o1 * CAP

    def per_chunk(ch, carry):
        c0, d0, c1, d1 = carry
        pltpu.sync_copy(src_hbm.at[pl.ds(ch * CE, CE)], src_v)
        pltpu.sync_copy(dst_hbm.at[pl.ds(ch * CE, CE)], dst_v)

        def per_group(i, gc):
            g0, gd0, g1, gd1 = gc
            s16 = src_v[pl.ds(i * LANE, LANE)]
            d16 = dst_v[pl.ds(i * LANE, LANE)]
            e16 = ch * CE + i * LANE + ii
            dl0 = d16 - ob0
            m0 = (dl0 >= 0) & (dl0 < TR)
            cs0 = plsc.cumsum(m0.astype(jnp.int32))
            pos0 = g0 + cs0 - 1
            plsc.store_scatter(bs0, [pos0], s16 * 256 + dl0, mask=m0)
            plsc.store_scatter(be0, [pos0], e16 * 256 + dl0, mask=m0)
            g0 = g0 + cs0[15]
            dl1 = d16 - ob1
            m1 = (dl1 >= 0) & (dl1 < TR)
            cs1 = plsc.cumsum(m1.astype(jnp.int32))
            pos1 = g1 + cs1 - 1
            plsc.store_scatter(bs1, [pos1], s16 * 256 + dl1, mask=m1)
            plsc.store_scatter(be1, [pos1], e16 * 256 + dl1, mask=m1)
            g1 = g1 + cs1[15]
            g0, gd0 = _flush_ready(bs0, be0, ssrc_hbm, seid_hbm, sb0 + PB, g0, gd0)
            g1, gd1 = _flush_ready(bs1, be1, ssrc_hbm, seid_hbm, sb1 + PB, g1, gd1)
            return (g0, gd0, g1, gd1)

        return lax.fori_loop(0, GROUPS, per_group, (c0, d0, c1, d1))

    z = jnp.int32(0)
    c0, d0, c1, d1 = lax.fori_loop(0, NCHUNK, per_chunk, (z, z, z, z))

    def finish(buf_s, buf_e, sbase, cnt, doff):
        for j in range(PB // LANE):
            plsc.store_scatter(buf_s, [cnt + j * LANE + ii],
                               jnp.full((LANE,), N * 256, jnp.int32))
            plsc.store_scatter(buf_e, [cnt + j * LANE + ii],
                               jnp.full((LANE,), E * 256, jnp.int32))
        nb = (cnt + PB - 1) // PB

        def wr(j, _):
            off = pl.multiple_of(sbase + PB + doff + j * PB, 8)
            pltpu.sync_copy(buf_s.at[pl.ds(j * PB, PB)],
                            ssrc_hbm.at[pl.ds(off, PB)])
            pltpu.sync_copy(buf_e.at[pl.ds(j * PB, PB)],
                            seid_hbm.at[pl.ds(off, PB)])
            return 0

        lax.fori_loop(0, nb, wr, 0)
        slen = doff + nb * PB
        hdr = jnp.where(ii == 0, slen * 256, N * 256)
        plsc.store_scatter(buf_s, [ii], hdr)
        plsc.store_scatter(buf_e, [ii], jnp.where(ii == 0, slen * 256, E * 256))
        for j in range(1, PB // LANE):
            plsc.store_scatter(buf_s, [j * LANE + ii],
                               jnp.full((LANE,), N * 256, jnp.int32))
            plsc.store_scatter(buf_e, [j * LANE + ii],
                               jnp.full((LANE,), E * 256, jnp.int32))
        pltpu.sync_copy(buf_s.at[pl.ds(0, PB)], ssrc_hbm.at[pl.ds(sbase, PB)])
        pltpu.sync_copy(buf_e.at[pl.ds(0, PB)], seid_hbm.at[pl.ds(sbase, PB)])

    finish(bs0, be0, sb0, c0, d0)
    finish(bs1, be1, sb1, c1, d1)


_strips = pl.kernel(
    _strips_body,
    out_type=(
        jax.ShapeDtypeStruct((NOWN * CAP + PCAP,), jnp.int32),
        jax.ShapeDtypeStruct((NOWN * CAP + PCAP,), jnp.int32),
    ),
    mesh=plsc.VectorSubcoreMesh(core_axis_name="c", subcore_axis_name="s",
                                num_cores=NC, num_subcores=NS),
    compiler_params=pltpu.CompilerParams(needs_layout_passes=False),
    scratch_types=[
        pltpu.VMEM((CE,), jnp.int32),
        pltpu.VMEM((CE,), jnp.int32),
        pltpu.VMEM((BUFCAP,), jnp.int32),
        pltpu.VMEM((BUFCAP,), jnp.int32),
        pltpu.VMEM((BUFCAP,), jnp.int32),
        pltpu.VMEM((BUFCAP,), jnp.int32),
    ],
)


def _pass_body(table, strip_hbm, zeros_hbm, out,
               pair_ch, hdr_v, gidx_a, gidx_b, dl_a, dl_b,
               rows_a, rows_b, acc, sem_a, sem_b):
    core = lax.axis_index("c")
    sub = lax.axis_index("s")
    wid = sub * NC + core
    ii = lax.iota(jnp.int32, LANE)

    def fetch(j, sbase, gidx_v, dl_v):
        @pl.when(j % NBC == 0)
        def _refill():
            off = pl.multiple_of(sbase + PB + j * PB, 8)
            pltpu.sync_copy(strip_hbm.at[pl.ds(off, PCAP)], pair_ch)

        local = pl.multiple_of((j % NBC) * PB, 8)
        for q in range(PB // LANE):
            v = pair_ch[pl.ds(local + q * LANE, LANE)]
            gidx_v[pl.ds(q * LANE, LANE)] = v >> 8
            dl_v[pl.ds(q * LANE, LANE)] = v & 255

    def accrow(dl_v, rows_v):
        d0 = dl_v[pl.ds(0, LANE)] * D2
        d1 = dl_v[pl.ds(LANE, LANE)] * D2

        def one_row(k, dsel):
            km_splat = jnp.zeros((LANE,), jnp.int32) + (k % LANE)
            base = dsel.at[km_splat].get(mode="promise_in_bounds") + ii
            for c in range(D2 // LANE):
                vals = rows_v[k, pl.ds(c * LANE, LANE)]
                plsc.addupdate_scatter(acc, [base + c * LANE], vals)

        def body(k2, _):
            k = 2 * k2
            dsel = jnp.where(k < LANE, d0, d1)
            one_row(k, dsel)
            dsel1 = jnp.where(k + 1 < LANE, d0, d1)
            one_row(k + 1, dsel1)
            return 0

        lax.fori_loop(0, PB // 2, body, 0)

    for r in range(ROUNDS):
        o = r * NW + wid
        obase = o * TR
        sbase = o * CAP
        pltpu.sync_copy(zeros_hbm, acc)
        pltpu.sync_copy(strip_hbm.at[pl.ds(sbase, PB)], hdr_v)
        hv = hdr_v[pl.ds(0, LANE)]
        nb = (hv[0] >> 8) // PB

        @pl.when(nb > 0)
        def _prime():
            fetch(0, sbase, gidx_a, dl_a)
            pltpu.async_copy(table.at[gidx_a], rows_a, sem_a)

        def body2(i, _):
            j1 = 2 * i + 1

            @pl.when(j1 < nb)
            def _fire_b():
                fetch(j1, sbase, gidx_b, dl_b)
                pltpu.async_copy(table.at[gidx_b], rows_b, sem_b)

            pltpu.make_async_copy(table.at[gidx_a], rows_a, sem_a).wait()
            accrow(dl_a, rows_a)

            @pl.when(j1 + 1 < nb)
            def _fire_a():
                fetch(j1 + 1, sbase, gidx_a, dl_a)
                pltpu.async_copy(table.at[gidx_a], rows_a, sem_a)

            @pl.when(j1 < nb)
            def _drain_b():
                pltpu.make_async_copy(table.at[gidx_b], rows_b, sem_b).wait()
                accrow(dl_b, rows_b)

            return 0

        lax.fori_loop(0, (nb + 1) // 2, body2, 0)
        pltpu.sync_copy(acc, out.at[pl.ds(obase * D2, TR * D2)])


def _make_pass():
    return pl.kernel(
        _pass_body,
        out_type=jax.ShapeDtypeStruct((NPAD * D2,), jnp.float32),
        mesh=plsc.VectorSubcoreMesh(core_axis_name="c", subcore_axis_name="s",
                                    num_cores=NC, num_subcores=NS),
        compiler_params=pltpu.CompilerParams(needs_layout_passes=False),
        scratch_types=[
            pltpu.VMEM((PCAP,), jnp.int32),
            pltpu.VMEM((PB,), jnp.int32),
            pltpu.VMEM((PB,), jnp.int32),
            pltpu.VMEM((PB,), jnp.int32),
            pltpu.VMEM((PB,), jnp.int32),
            pltpu.VMEM((PB,), jnp.int32),
            pltpu.VMEM((PB, D2), jnp.float32),
            pltpu.VMEM((PB, D2), jnp.float32),
            pltpu.VMEM((TR * D2,), jnp.float32),
            pltpu.SemaphoreType.DMA,
            pltpu.SemaphoreType.DMA,
        ],
    )


_sc_pass = _make_pass()


BM = 2504
BMD = 1000


def _layer_body(relu_out, h_ref, c_ref, a_ref, w1_ref, b1_ref, w2_ref, b2_ref,
                o_ref):
    z = h_ref[...] + c_ref[...] + a_ref[...]
    z = jnp.dot(z, w1_ref[...], preferred_element_type=jnp.float32) + b1_ref[...]
    z = jnp.maximum(z, 0.0)
    z = jnp.dot(z, w2_ref[...], preferred_element_type=jnp.float32) + b2_ref[...]
    if relu_out:
        z = jnp.maximum(z, 0.0)
    rows = pl.program_id(0) * BM + lax.broadcasted_iota(jnp.int32, (BM, 1), 0)
    o_ref[...] = jnp.where(rows < N, z, 0.0)


def _make_layer(relu_out):
    return pl.pallas_call(
        functools.partial(_layer_body, relu_out),
        grid=(TN // BM,),
        in_specs=[
            pl.BlockSpec((BM, D2), lambda i: (i, 0)),
            pl.BlockSpec((BM, D2), lambda i: (i, 0)),
            pl.BlockSpec((BM, D2), lambda i: (i, 0)),
            pl.BlockSpec((D2, D2), lambda i: (0, 0)),
            pl.BlockSpec((1, D2), lambda i: (0, 0)),
            pl.BlockSpec((D2, D2), lambda i: (0, 0)),
            pl.BlockSpec((1, D2), lambda i: (0, 0)),
        ],
        out_specs=pl.BlockSpec((BM, D2), lambda i: (i, 0)),
        out_shape=jax.ShapeDtypeStruct((TN, D2), jnp.float32),
    )


def _n2d_body(h_ref, keep_ref, w_ref, a_ref, o_ref):
    h = h_ref[...]
    a = a_ref[0, 0]
    z = jnp.where(h >= 0.0, h, a * h)
    z = jnp.dot(z, w_ref[...], preferred_element_type=jnp.float32)
    o_ref[...] = z * keep_ref[...]


_n2d = pl.pallas_call(
    _n2d_body,
    grid=(TN // BM,),
    in_specs=[
        pl.BlockSpec((BM, D2), lambda i: (i, 0)),
        pl.BlockSpec((BM, 1), lambda i: (i, 0)),
        pl.BlockSpec((D2, D2), lambda i: (0, 0)),
        pl.BlockSpec((1, 1), lambda i: (0, 0)),
    ],
    out_specs=pl.BlockSpec((BM, D2), lambda i: (i, 0)),
    out_shape=jax.ShapeDtypeStruct((TN, D2), jnp.float32),
)


def _dec_body(h_ref, c_ref, a_ref, w_ref, b_ref, o_ref):
    z = h_ref[...] + c_ref[...] + a_ref[...]
    o_ref[...] = jnp.dot(z, w_ref[...], preferred_element_type=jnp.float32) + b_ref[...]


_dec = pl.pallas_call(
    _dec_body,
    grid=(N // BMD,),
    in_specs=[
        pl.BlockSpec((BMD, D2), lambda i: (i, 0)),
        pl.BlockSpec((BMD, D2), lambda i: (i, 0)),
        pl.BlockSpec((BMD, D2), lambda i: (i, 0)),
        pl.BlockSpec((D2, OUT), lambda i: (0, 0)),
        pl.BlockSpec((1, OUT), lambda i: (0, 0)),
    ],
    out_specs=pl.BlockSpec((BMD, OUT), lambda i: (i, 0)),
    out_shape=jax.ShapeDtypeStruct((N, OUT), jnp.float32),
)

_layer_mid = _make_layer(True)
_layer_last = _make_layer(False)


def kernel(x, edge_index, edge_attr, masked_atom_mask, enc_W1, enc_b1,
           enc_W2, enc_b2, prelu_a, W_n2d, dec_W, dec_b):
    src = edge_index[0]
    dst = edge_index[1]
    x_pad = jnp.pad(x, ((0, TN - N), (0, D2 - D)))
    ea_pad = jnp.pad(edge_attr, ((0, TE - E), (0, D2 - D)))
    w1p = jnp.pad(enc_W1, ((0, 0), (0, D2 - D), (0, D2 - D)))
    b1p = jnp.pad(enc_b1, ((0, 0), (0, D2 - D)))
    w2p = jnp.pad(enc_W2, ((0, 0), (0, D2 - D), (0, D2 - D)))
    b2p = jnp.pad(enc_b2, ((0, 0), (0, D2 - D)))
    wn2dp = jnp.pad(W_n2d, ((0, D2 - D), (0, D2 - D)))
    dec_wp = jnp.pad(dec_W, ((0, D2 - D), (0, 0)))
    keep = jnp.pad(1.0 - masked_atom_mask.astype(jnp.float32).reshape(N, 1),
                   ((0, TN - N), (0, 0)))
    zeros_stage = jnp.zeros((TR * D2,), jnp.float32)

    ssrc, seid = _strips(src, dst)

    C = _sc_pass(ea_pad, seid, zeros_stage).reshape(NPAD, D2)

    h = x_pad
    for l in range(L):
        agg = _sc_pass(h, ssrc, zeros_stage).reshape(NPAD, D2)
        layer = _layer_mid if l < L - 1 else _layer_last
        h = layer(h, C, agg, w1p[l], b1p[l].reshape(1, D2),
                  w2p[l], b2p[l].reshape(1, D2))

    h = _n2d(h, keep, wn2dp, prelu_a.reshape(1, 1))

    agg = _sc_pass(h, ssrc, zeros_stage).reshape(NPAD, D2)
    return _dec(h, C, agg, dec_wp, dec_b.reshape(1, OUT))


# speedup vs baseline: 3.7350x; 1.7106x over previous
import functools

import jax
import jax.numpy as jnp
from jax import lax
from jax.experimental import pallas as pl
from jax.experimental.pallas import tpu as pltpu
from jax.experimental.pallas import tpu_sc as plsc

N = 10000
E = 160000
D = 500
OUT = 119
L = 5

D2 = 512
TN = N + 16
TE = E + 16
NC = 2
NS = 16
NW = NC * NS
LANE = 16

TR = 160
ROUNDS = 2
NPAD = ROUNDS * NW * TR
NOWN = ROUNDS * NW
PB = 32
CE = 10000
NCHUNK = E // CE
GROUPS = CE // LANE
FB = 960
BUFCAP = 1024
CAP = PB + E + PB // 1
PCAP = 6144
NBC = PCAP // PB


def _flush_ready(buf_s, buf_e, strip_s, strip_e, sbase, cnt, doff):
    def do_flush(args):
        c, d = args
        off = pl.multiple_of(sbase + d, 8)
        pltpu.sync_copy(buf_s.at[pl.ds(0, FB)], strip_s.at[pl.ds(off, FB)])
        pltpu.sync_copy(buf_e.at[pl.ds(0, FB)], strip_e.at[pl.ds(off, FB)])
        ii = lax.iota(jnp.int32, LANE)
        resid = c - FB
        vs = buf_s[pl.ds(FB, LANE)]
        ve = buf_e[pl.ds(FB, LANE)]
        plsc.store_scatter(buf_s, [ii], vs, mask=ii < resid)
        plsc.store_scatter(buf_e, [ii], ve, mask=ii < resid)
        return (resid, d + FB)

    return lax.cond(cnt >= FB, do_flush, lambda a: a, (cnt, doff))


def _strips_body(src_hbm, dst_hbm, ssrc_hbm, seid_hbm,
                 src_v, dst_v, bs0, be0, bs1, be1):
    core = lax.axis_index("c")
    sub = lax.axis_index("s")
    wid = sub * NC + core
    ii = lax.iota(jnp.int32, LANE)
    o0 = wid
    o1 = NW + wid
    ob0 = o0 * TR
    ob1 = o1 * TR
    sb0 = o0 * CAP
    sb1 = o1 * CAP

    def per_chunk(ch, carry):
        c0, d0, c1, d1 = carry
        pltpu.sync_copy(src_hbm.at[pl.ds(ch * CE, CE)], src_v)
        pltpu.sync_copy(dst_hbm.at[pl.ds(ch * CE, CE)], dst_v)

        def per_group(i, gc):
            g0, gd0, g1, gd1 = gc
            s16 = src_v[pl.ds(i * LANE, LANE)]
            d16 = dst_v[pl.ds(i * LANE, LANE)]
            e16 = ch * CE + i * LANE + ii
            dl0 = d16 - ob0
            m0 = (dl0 >= 0) & (dl0 < TR)
            cs0 = plsc.cumsum(m0.astype(jnp.int32))
            pos0 = g0 + cs0 - 1
            plsc.store_scatter(bs0, [pos0], s16 * 256 + dl0, mask=m0)
            plsc.store_scatter(be0, [pos0], e16 * 256 + dl0, mask=m0)
            g0 = g0 + cs0[15]
            dl1 = d16 - ob1
            m1 = (dl1 >= 0) & (dl1 < TR)
            cs1 = plsc.cumsum(m1.astype(jnp.int32))
            pos1 = g1 + cs1 - 1
            plsc.store_scatter(bs1, [pos1], s16 * 256 + dl1, mask=m1)
            plsc.store_scatter(be1, [pos1], e16 * 256 + dl1, mask=m1)
            g1 = g1 + cs1[15]
            g0, gd0 = _flush_ready(bs0, be0, ssrc_hbm, seid_hbm, sb0 + PB, g0, gd0)
            g1, gd1 = _flush_ready(bs1, be1, ssrc_hbm, seid_hbm, sb1 + PB, g1, gd1)
            return (g0, gd0, g1, gd1)

        return lax.fori_loop(0, GROUPS, per_group, (c0, d0, c1, d1))

    z = jnp.int32(0)
    c0, d0, c1, d1 = lax.fori_loop(0, NCHUNK, per_chunk, (z, z, z, z))

    def finish(buf_s, buf_e, sbase, cnt, doff):
        for j in range(PB // LANE):
            plsc.store_scatter(buf_s, [cnt + j * LANE + ii],
                               jnp.full((LANE,), N * 256, jnp.int32))
            plsc.store_scatter(buf_e, [cnt + j * LANE + ii],
                               jnp.full((LANE,), E * 256, jnp.int32))
        nb = (cnt + PB - 1) // PB

        def wr(j, _):
            off = pl.multiple_of(sbase + PB + doff + j * PB, 8)
            pltpu.sync_copy(buf_s.at[pl.ds(j * PB, PB)],
                            ssrc_hbm.at[pl.ds(off, PB)])
            pltpu.sync_copy(buf_e.at[pl.ds(j * PB, PB)],
                            seid_hbm.at[pl.ds(off, PB)])
            return 0

        lax.fori_loop(0, nb, wr, 0)
        slen = doff + nb * PB
        hdr = jnp.where(ii == 0, slen * 256, N * 256)
        plsc.store_scatter(buf_s, [ii], hdr)
        plsc.store_scatter(buf_e, [ii], jnp.where(ii == 0, slen * 256, E * 256))
        for j in range(1, PB // LANE):
            plsc.store_scatter(buf_s, [j * LANE + ii],
                               jnp.full((LANE,), N * 256, jnp.int32))
            plsc.store_scatter(buf_e, [j * LANE + ii],
                               jnp.full((LANE,), E * 256, jnp.int32))
        pltpu.sync_copy(buf_s.at[pl.ds(0, PB)], ssrc_hbm.at[pl.ds(sbase, PB)])
        pltpu.sync_copy(buf_e.at[pl.ds(0, PB)], seid_hbm.at[pl.ds(sbase, PB)])

    finish(bs0, be0, sb0, c0, d0)
    finish(bs1, be1, sb1, c1, d1)


_strips = pl.kernel(
    _strips_body,
    out_type=(
        jax.ShapeDtypeStruct((NOWN * CAP + PCAP,), jnp.int32),
        jax.ShapeDtypeStruct((NOWN * CAP + PCAP,), jnp.int32),
    ),
    mesh=plsc.VectorSubcoreMesh(core_axis_name="c", subcore_axis_name="s",
                                num_cores=NC, num_subcores=NS),
    compiler_params=pltpu.CompilerParams(needs_layout_passes=False),
    scratch_types=[
        pltpu.VMEM((CE,), jnp.int32),
        pltpu.VMEM((CE,), jnp.int32),
        pltpu.VMEM((BUFCAP,), jnp.int32),
        pltpu.VMEM((BUFCAP,), jnp.int32),
        pltpu.VMEM((BUFCAP,), jnp.int32),
        pltpu.VMEM((BUFCAP,), jnp.int32),
    ],
)


def _pass_body(table, strip_hbm, zeros_hbm, out,
               pair_ch, hdr_v, gidx_a, gidx_b, dl_a, dl_b,
               rows_a, rows_b, acc, sem_a, sem_b):
    core = lax.axis_index("c")
    sub = lax.axis_index("s")
    wid = sub * NC + core
    ii = lax.iota(jnp.int32, LANE)

    def fetch(j, sbase, gidx_v, dl_v):
        @pl.when(j % NBC == 0)
        def _refill():
            off = pl.multiple_of(sbase + PB + j * PB, 8)
            pltpu.sync_copy(strip_hbm.at[pl.ds(off, PCAP)], pair_ch)

        local = pl.multiple_of((j % NBC) * PB, 8)
        for q in range(PB // LANE):
            v = pair_ch[pl.ds(local + q * LANE, LANE)]
            gidx_v[pl.ds(q * LANE, LANE)] = v >> 8
            dl_v[pl.ds(q * LANE, LANE)] = v & 255

    def accrow(dl_v, rows_v):
        d0 = dl_v[pl.ds(0, LANE)] * D2
        d1 = dl_v[pl.ds(LANE, LANE)] * D2

        def one_row(k, dsel):
            km_splat = jnp.zeros((LANE,), jnp.int32) + (k % LANE)
            base = dsel.at[km_splat].get(mode="promise_in_bounds") + ii
            for g in range(D2 // LANE // 4):
                vals = [rows_v[k, pl.ds((4 * g + u) * LANE, LANE)]
                        for u in range(4)]
                for u in range(4):
                    plsc.addupdate_scatter(acc, [base + (4 * g + u) * LANE],
                                           vals[u])

        def body(k2, _):
            k = 2 * k2
            dsel = jnp.where(k < LANE, d0, d1)
            one_row(k, dsel)
            dsel1 = jnp.where(k + 1 < LANE, d0, d1)
            one_row(k + 1, dsel1)
            return 0

        lax.fori_loop(0, PB // 2, body, 0)

    for r in range(ROUNDS):
        o = r * NW + wid
        obase = o * TR
        sbase = o * CAP
        pltpu.sync_copy(zeros_hbm, acc)
        pltpu.sync_copy(strip_hbm.at[pl.ds(sbase, PB)], hdr_v)
        hv = hdr_v[pl.ds(0, LANE)]
        nb = (hv[0] >> 8) // PB

        @pl.when(nb > 0)
        def _prime():
            fetch(0, sbase, gidx_a, dl_a)
            pltpu.async_copy(table.at[gidx_a], rows_a, sem_a)

        def body2(i, _):
            j1 = 2 * i + 1

            @pl.when(j1 < nb)
            def _fire_b():
                fetch(j1, sbase, gidx_b, dl_b)
                pltpu.async_copy(table.at[gidx_b], rows_b, sem_b)

            pltpu.make_async_copy(table.at[gidx_a], rows_a, sem_a).wait()
            accrow(dl_a, rows_a)

            @pl.when(j1 + 1 < nb)
            def _fire_a():
                fetch(j1 + 1, sbase, gidx_a, dl_a)
                pltpu.async_copy(table.at[gidx_a], rows_a, sem_a)

            @pl.when(j1 < nb)
            def _drain_b():
                pltpu.make_async_copy(table.at[gidx_b], rows_b, sem_b).wait()
                accrow(dl_b, rows_b)

            return 0

        lax.fori_loop(0, (nb + 1) // 2, body2, 0)
        pltpu.sync_copy(acc, out.at[pl.ds(obase * D2, TR * D2)])


def _make_pass():
    return pl.kernel(
        _pass_body,
        out_type=jax.ShapeDtypeStruct((NPAD * D2,), jnp.float32),
        mesh=plsc.VectorSubcoreMesh(core_axis_name="c", subcore_axis_name="s",
                                    num_cores=NC, num_subcores=NS),
        compiler_params=pltpu.CompilerParams(needs_layout_passes=False),
        scratch_types=[
            pltpu.VMEM((PCAP,), jnp.int32),
            pltpu.VMEM((PB,), jnp.int32),
            pltpu.VMEM((PB,), jnp.int32),
            pltpu.VMEM((PB,), jnp.int32),
            pltpu.VMEM((PB,), jnp.int32),
            pltpu.VMEM((PB,), jnp.int32),
            pltpu.VMEM((PB, D2), jnp.float32),
            pltpu.VMEM((PB, D2), jnp.float32),
            pltpu.VMEM((TR * D2,), jnp.float32),
            pltpu.SemaphoreType.DMA,
            pltpu.SemaphoreType.DMA,
        ],
    )


_sc_pass = _make_pass()


BM = 2504
BMD = 1000


def _layer_body(relu_out, h_ref, c_ref, a_ref, w1_ref, b1_ref, w2_ref, b2_ref,
                o_ref):
    z = h_ref[...] + c_ref[...] + a_ref[...]
    z = jnp.dot(z, w1_ref[...], preferred_element_type=jnp.float32) + b1_ref[...]
    z = jnp.maximum(z, 0.0)
    z = jnp.dot(z, w2_ref[...], preferred_element_type=jnp.float32) + b2_ref[...]
    if relu_out:
        z = jnp.maximum(z, 0.0)
    rows = pl.program_id(0) * BM + lax.broadcasted_iota(jnp.int32, (BM, 1), 0)
    o_ref[...] = jnp.where(rows < N, z, 0.0)


def _make_layer(relu_out):
    return pl.pallas_call(
        functools.partial(_layer_body, relu_out),
        grid=(TN // BM,),
        in_specs=[
            pl.BlockSpec((BM, D2), lambda i: (i, 0)),
            pl.BlockSpec((BM, D2), lambda i: (i, 0)),
            pl.BlockSpec((BM, D2), lambda i: (i, 0)),
            pl.BlockSpec((D2, D2), lambda i: (0, 0)),
            pl.BlockSpec((1, D2), lambda i: (0, 0)),
            pl.BlockSpec((D2, D2), lambda i: (0, 0)),
            pl.BlockSpec((1, D2), lambda i: (0, 0)),
        ],
        out_specs=pl.BlockSpec((BM, D2), lambda i: (i, 0)),
        out_shape=jax.ShapeDtypeStruct((TN, D2), jnp.float32),
    )


def _n2d_body(h_ref, keep_ref, w_ref, a_ref, o_ref):
    h = h_ref[...]
    a = a_ref[0, 0]
    z = jnp.where(h >= 0.0, h, a * h)
    z = jnp.dot(z, w_ref[...], preferred_element_type=jnp.float32)
    o_ref[...] = z * keep_ref[...]


_n2d = pl.pallas_call(
    _n2d_body,
    grid=(TN // BM,),
    in_specs=[
        pl.BlockSpec((BM, D2), lambda i: (i, 0)),
        pl.BlockSpec((BM, 1), lambda i: (i, 0)),
        pl.BlockSpec((D2, D2), lambda i: (0, 0)),
        pl.BlockSpec((1, 1), lambda i: (0, 0)),
    ],
    out_specs=pl.BlockSpec((BM, D2), lambda i: (i, 0)),
    out_shape=jax.ShapeDtypeStruct((TN, D2), jnp.float32),
)


def _dec_body(h_ref, c_ref, a_ref, w_ref, b_ref, o_ref):
    z = h_ref[...] + c_ref[...] + a_ref[...]
    o_ref[...] = jnp.dot(z, w_ref[...], preferred_element_type=jnp.float32) + b_ref[...]


_dec = pl.pallas_call(
    _dec_body,
    grid=(N // BMD,),
    in_specs=[
        pl.BlockSpec((BMD, D2), lambda i: (i, 0)),
        pl.BlockSpec((BMD, D2), lambda i: (i, 0)),
        pl.BlockSpec((BMD, D2), lambda i: (i, 0)),
        pl.BlockSpec((D2, OUT), lambda i: (0, 0)),
        pl.BlockSpec((1, OUT), lambda i: (0, 0)),
    ],
    out_specs=pl.BlockSpec((BMD, OUT), lambda i: (i, 0)),
    out_shape=jax.ShapeDtypeStruct((N, OUT), jnp.float32),
)

_layer_mid = _make_layer(True)
_layer_last = _make_layer(False)


def kernel(x, edge_index, edge_attr, masked_atom_mask, enc_W1, enc_b1,
           enc_W2, enc_b2, prelu_a, W_n2d, dec_W, dec_b):
    src = edge_index[0]
    dst = edge_index[1]
    x_pad = jnp.pad(x, ((0, TN - N), (0, D2 - D)))
    ea_pad = jnp.pad(edge_attr, ((0, TE - E), (0, D2 - D)))
    w1p = jnp.pad(enc_W1, ((0, 0), (0, D2 - D), (0, D2 - D)))
    b1p = jnp.pad(enc_b1, ((0, 0), (0, D2 - D)))
    w2p = jnp.pad(enc_W2, ((0, 0), (0, D2 - D), (0, D2 - D)))
    b2p = jnp.pad(enc_b2, ((0, 0), (0, D2 - D)))
    wn2dp = jnp.pad(W_n2d, ((0, D2 - D), (0, D2 - D)))
    dec_wp = jnp.pad(dec_W, ((0, D2 - D), (0, 0)))
    keep = jnp.pad(1.0 - masked_atom_mask.astype(jnp.float32).reshape(N, 1),
                   ((0, TN - N), (0, 0)))
    zeros_stage = jnp.zeros((TR * D2,), jnp.float32)

    ssrc, seid = _strips(src, dst)

    C = _sc_pass(ea_pad, seid, zeros_stage).reshape(NPAD, D2)

    h = x_pad
    for l in range(L):
        agg = _sc_pass(h, ssrc, zeros_stage).reshape(NPAD, D2)
        layer = _layer_mid if l < L - 1 else _layer_last
        h = layer(h, C, agg, w1p[l], b1p[l].reshape(1, D2),
                  w2p[l], b2p[l].reshape(1, D2))

    h = _n2d(h, keep, wn2dp, prelu_a.reshape(1, 1))

    agg = _sc_pass(h, ssrc, zeros_stage).reshape(NPAD, D2)
    return _dec(h, C, agg, dec_wp, dec_b.reshape(1, OUT))


# speedup vs baseline: 3.8326x; 1.0261x over previous
import functools

import jax
import jax.numpy as jnp
from jax import lax
from jax.experimental import pallas as pl
from jax.experimental.pallas import tpu as pltpu
from jax.experimental.pallas import tpu_sc as plsc

N = 10000
E = 160000
D = 500
OUT = 119
L = 5

D2 = 512
TN = N + 16
TE = E + 16
NC = 2
NS = 16
NW = NC * NS
LANE = 16

TR = 160
ROUNDS = 2
NPAD = ROUNDS * NW * TR
NOWN = ROUNDS * NW
PB = 32
CE = 10000
NCHUNK = E // CE
GROUPS = CE // LANE
FB = 960
BUFCAP = 1024
CAP = PB + E + PB // 1
PCAP = 6144
NBC = PCAP // PB


def _flush_ready(buf_s, buf_e, strip_s, strip_e, sbase, cnt, doff):
    def do_flush(args):
        c, d = args
        off = pl.multiple_of(sbase + d, 8)
        pltpu.sync_copy(buf_s.at[pl.ds(0, FB)], strip_s.at[pl.ds(off, FB)])
        pltpu.sync_copy(buf_e.at[pl.ds(0, FB)], strip_e.at[pl.ds(off, FB)])
        ii = lax.iota(jnp.int32, LANE)
        resid = c - FB
        for q in range(2):
            vs = buf_s[pl.ds(FB + q * LANE, LANE)]
            ve = buf_e[pl.ds(FB + q * LANE, LANE)]
            lanes = q * LANE + ii
            plsc.store_scatter(buf_s, [lanes], vs, mask=lanes < resid)
            plsc.store_scatter(buf_e, [lanes], ve, mask=lanes < resid)
        return (resid, d + FB)

    return lax.cond(cnt >= FB, do_flush, lambda a: a, (cnt, doff))


def _strips_body(src_hbm, dst_hbm, ssrc_hbm, seid_hbm,
                 src_v, dst_v, bs0, be0, bs1, be1):
    core = lax.axis_index("c")
    sub = lax.axis_index("s")
    wid = sub * NC + core
    ii = lax.iota(jnp.int32, LANE)
    o0 = wid
    o1 = NW + wid
    ob0 = o0 * TR
    ob1 = o1 * TR
    sb0 = o0 * CAP
    sb1 = o1 * CAP

    def per_chunk(ch, carry):
        c0, d0, c1, d1 = carry
        pltpu.sync_copy(src_hbm.at[pl.ds(ch * CE, CE)], src_v)
        pltpu.sync_copy(dst_hbm.at[pl.ds(ch * CE, CE)], dst_v)

        def one_group(i, g0, g1):
            s16 = src_v[pl.ds(i * LANE, LANE)]
            d16 = dst_v[pl.ds(i * LANE, LANE)]
            e16 = ch * CE + i * LANE + ii
            dl0 = d16 - ob0
            m0 = (dl0 >= 0) & (dl0 < TR)
            dl1 = d16 - ob1
            m1 = (dl1 >= 0) & (dl1 < TR)
            cs0 = plsc.cumsum(m0.astype(jnp.int32))
            cs1 = plsc.cumsum(m1.astype(jnp.int32))
            pos0 = g0 + cs0 - 1
            plsc.store_scatter(bs0, [pos0], s16 * 256 + dl0, mask=m0)
            plsc.store_scatter(be0, [pos0], e16 * 256 + dl0, mask=m0)
            pos1 = g1 + cs1 - 1
            plsc.store_scatter(bs1, [pos1], s16 * 256 + dl1, mask=m1)
            plsc.store_scatter(be1, [pos1], e16 * 256 + dl1, mask=m1)
            return g0 + cs0[15], g1 + cs1[15]

        def per_group2(i2, gc):
            g0, gd0, g1, gd1 = gc
            g0, g1 = one_group(2 * i2, g0, g1)
            g0, g1 = one_group(2 * i2 + 1, g0, g1)
            g0, gd0 = _flush_ready(bs0, be0, ssrc_hbm, seid_hbm, sb0 + PB, g0, gd0)
            g1, gd1 = _flush_ready(bs1, be1, ssrc_hbm, seid_hbm, sb1 + PB, g1, gd1)
            return (g0, gd0, g1, gd1)

        gc = lax.fori_loop(0, GROUPS // 2, per_group2, (c0, d0, c1, d1))
        if GROUPS % 2:
            g0, gd0, g1, gd1 = gc
            g0, g1 = one_group(GROUPS - 1, g0, g1)
            g0, gd0 = _flush_ready(bs0, be0, ssrc_hbm, seid_hbm, sb0 + PB, g0, gd0)
            g1, gd1 = _flush_ready(bs1, be1, ssrc_hbm, seid_hbm, sb1 + PB, g1, gd1)
            gc = (g0, gd0, g1, gd1)
        return gc

    z = jnp.int32(0)
    c0, d0, c1, d1 = lax.fori_loop(0, NCHUNK, per_chunk, (z, z, z, z))

    def finish(buf_s, buf_e, sbase, cnt, doff):
        for j in range(PB // LANE):
            plsc.store_scatter(buf_s, [cnt + j * LANE + ii],
                               jnp.full((LANE,), N * 256, jnp.int32))
            plsc.store_scatter(buf_e, [cnt + j * LANE + ii],
                               jnp.full((LANE,), E * 256, jnp.int32))
        nb = (cnt + PB - 1) // PB

        def wr(j, _):
            off = pl.multiple_of(sbase + PB + doff + j * PB, 8)
            pltpu.sync_copy(buf_s.at[pl.ds(j * PB, PB)],
                            ssrc_hbm.at[pl.ds(off, PB)])
            pltpu.sync_copy(buf_e.at[pl.ds(j * PB, PB)],
                            seid_hbm.at[pl.ds(off, PB)])
            return 0

        lax.fori_loop(0, nb, wr, 0)
        slen = doff + nb * PB
        hdr = jnp.where(ii == 0, slen * 256, N * 256)
        plsc.store_scatter(buf_s, [ii], hdr)
        plsc.store_scatter(buf_e, [ii], jnp.where(ii == 0, slen * 256, E * 256))
        for j in range(1, PB // LANE):
            plsc.store_scatter(buf_s, [j * LANE + ii],
                               jnp.full((LANE,), N * 256, jnp.int32))
            plsc.store_scatter(buf_e, [j * LANE + ii],
                               jnp.full((LANE,), E * 256, jnp.int32))
        pltpu.sync_copy(buf_s.at[pl.ds(0, PB)], ssrc_hbm.at[pl.ds(sbase, PB)])
        pltpu.sync_copy(buf_e.at[pl.ds(0, PB)], seid_hbm.at[pl.ds(sbase, PB)])

    finish(bs0, be0, sb0, c0, d0)
    finish(bs1, be1, sb1, c1, d1)


_strips = pl.kernel(
    _strips_body,
    out_type=(
        jax.ShapeDtypeStruct((NOWN * CAP + PCAP,), jnp.int32),
        jax.ShapeDtypeStruct((NOWN * CAP + PCAP,), jnp.int32),
    ),
    mesh=plsc.VectorSubcoreMesh(core_axis_name="c", subcore_axis_name="s",
                                num_cores=NC, num_subcores=NS),
    compiler_params=pltpu.CompilerParams(needs_layout_passes=False),
    scratch_types=[
        pltpu.VMEM((CE,), jnp.int32),
        pltpu.VMEM((CE,), jnp.int32),
        pltpu.VMEM((BUFCAP,), jnp.int32),
        pltpu.VMEM((BUFCAP,), jnp.int32),
        pltpu.VMEM((BUFCAP,), jnp.int32),
        pltpu.VMEM((BUFCAP,), jnp.int32),
    ],
)


def _pass_body(table, strip_hbm, zeros_hbm, out,
               pair_ch, hdr_v, gidx_a, gidx_b, dl_a, dl_b,
               rows_a, rows_b, acc, sem_a, sem_b):
    core = lax.axis_index("c")
    sub = lax.axis_index("s")
    wid = sub * NC + core
    ii = lax.iota(jnp.int32, LANE)

    def fetch(j, sbase, gidx_v, dl_v):
        @pl.when(j % NBC == 0)
        def _refill():
            off = pl.multiple_of(sbase + PB + j * PB, 8)
            pltpu.sync_copy(strip_hbm.at[pl.ds(off, PCAP)], pair_ch)

        local = pl.multiple_of((j % NBC) * PB, 8)
        for q in range(PB // LANE):
            v = pair_ch[pl.ds(local + q * LANE, LANE)]
            gidx_v[pl.ds(q * LANE, LANE)] = v >> 8
            dl_v[pl.ds(q * LANE, LANE)] = v & 255

    def accrow(dl_v, rows_v):
        d0 = dl_v[pl.ds(0, LANE)] * D2
        d1 = dl_v[pl.ds(LANE, LANE)] * D2

        def one_row(k, dsel):
            km_splat = jnp.zeros((LANE,), jnp.int32) + (k % LANE)
            base = dsel.at[km_splat].get(mode="promise_in_bounds") + ii
            for g in range(D2 // LANE // 4):
                vals = [rows_v[k, pl.ds((4 * g + u) * LANE, LANE)]
                        for u in range(4)]
                for u in range(4):
                    plsc.addupdate_scatter(acc, [base + (4 * g + u) * LANE],
                                           vals[u])

        def body(k2, _):
            k = 2 * k2
            dsel = jnp.where(k < LANE, d0, d1)
            one_row(k, dsel)
            dsel1 = jnp.where(k + 1 < LANE, d0, d1)
            one_row(k + 1, dsel1)
            return 0

        lax.fori_loop(0, PB // 2, body, 0)

    for r in range(ROUNDS):
        o = r * NW + wid
        obase = o * TR
        sbase = o * CAP
        pltpu.sync_copy(zeros_hbm, acc)
        pltpu.sync_copy(strip_hbm.at[pl.ds(sbase, PB)], hdr_v)
        hv = hdr_v[pl.ds(0, LANE)]
        nb = (hv[0] >> 8) // PB

        @pl.when(nb > 0)
        def _prime():
            fetch(0, sbase, gidx_a, dl_a)
            pltpu.async_copy(table.at[gidx_a], rows_a, sem_a)

        def body2(i, _):
            j1 = 2 * i + 1

            @pl.when(j1 < nb)
            def _fire_b():
                fetch(j1, sbase, gidx_b, dl_b)
                pltpu.async_copy(table.at[gidx_b], rows_b, sem_b)

            pltpu.make_async_copy(table.at[gidx_a], rows_a, sem_a).wait()
            accrow(dl_a, rows_a)

            @pl.when(j1 + 1 < nb)
            def _fire_a():
                fetch(j1 + 1, sbase, gidx_a, dl_a)
                pltpu.async_copy(table.at[gidx_a], rows_a, sem_a)

            @pl.when(j1 < nb)
            def _drain_b():
                pltpu.make_async_copy(table.at[gidx_b], rows_b, sem_b).wait()
                accrow(dl_b, rows_b)

            return 0

        lax.fori_loop(0, (nb + 1) // 2, body2, 0)
        pltpu.sync_copy(acc, out.at[pl.ds(obase * D2, TR * D2)])


def _make_pass():
    return pl.kernel(
        _pass_body,
        out_type=jax.ShapeDtypeStruct((NPAD * D2,), jnp.float32),
        mesh=plsc.VectorSubcoreMesh(core_axis_name="c", subcore_axis_name="s",
                                    num_cores=NC, num_subcores=NS),
        compiler_params=pltpu.CompilerParams(needs_layout_passes=False),
        scratch_types=[
            pltpu.VMEM((PCAP,), jnp.int32),
            pltpu.VMEM((PB,), jnp.int32),
            pltpu.VMEM((PB,), jnp.int32),
            pltpu.VMEM((PB,), jnp.int32),
            pltpu.VMEM((PB,), jnp.int32),
            pltpu.VMEM((PB,), jnp.int32),
            pltpu.VMEM((PB, D2), jnp.float32),
            pltpu.VMEM((PB, D2), jnp.float32),
            pltpu.VMEM((TR * D2,), jnp.float32),
            pltpu.SemaphoreType.DMA,
            pltpu.SemaphoreType.DMA,
        ],
    )


_sc_pass = _make_pass()


BM = 2504
BMD = 1000


def _layer_body(relu_out, h_ref, c_ref, a_ref, w1_ref, b1_ref, w2_ref, b2_ref,
                o_ref):
    z = h_ref[...] + c_ref[...] + a_ref[...]
    z = jnp.dot(z, w1_ref[...], preferred_element_type=jnp.float32) + b1_ref[...]
    z = jnp.maximum(z, 0.0)
    z = jnp.dot(z, w2_ref[...], preferred_element_type=jnp.float32) + b2_ref[...]
    if relu_out:
        z = jnp.maximum(z, 0.0)
    rows = pl.program_id(0) * BM + lax.broadcasted_iota(jnp.int32, (BM, 1), 0)
    o_ref[...] = jnp.where(rows < N, z, 0.0)


def _make_layer(relu_out):
    return pl.pallas_call(
        functools.partial(_layer_body, relu_out),
        grid=(TN // BM,),
        in_specs=[
            pl.BlockSpec((BM, D2), lambda i: (i, 0)),
            pl.BlockSpec((BM, D2), lambda i: (i, 0)),
            pl.BlockSpec((BM, D2), lambda i: (i, 0)),
            pl.BlockSpec((D2, D2), lambda i: (0, 0)),
            pl.BlockSpec((1, D2), lambda i: (0, 0)),
            pl.BlockSpec((D2, D2), lambda i: (0, 0)),
            pl.BlockSpec((1, D2), lambda i: (0, 0)),
        ],
        out_specs=pl.BlockSpec((BM, D2), lambda i: (i, 0)),
        out_shape=jax.ShapeDtypeStruct((TN, D2), jnp.float32),
    )


def _n2d_body(h_ref, keep_ref, w_ref, a_ref, o_ref):
    h = h_ref[...]
    a = a_ref[0, 0]
    z = jnp.where(h >= 0.0, h, a * h)
    z = jnp.dot(z, w_ref[...], preferred_element_type=jnp.float32)
    o_ref[...] = z * keep_ref[...]


_n2d = pl.pallas_call(
    _n2d_body,
    grid=(TN // BM,),
    in_specs=[
        pl.BlockSpec((BM, D2), lambda i: (i, 0)),
        pl.BlockSpec((BM, 1), lambda i: (i, 0)),
        pl.BlockSpec((D2, D2), lambda i: (0, 0)),
        pl.BlockSpec((1, 1), lambda i: (0, 0)),
    ],
    out_specs=pl.BlockSpec((BM, D2), lambda i: (i, 0)),
    out_shape=jax.ShapeDtypeStruct((TN, D2), jnp.float32),
)


def _dec_body(h_ref, c_ref, a_ref, w_ref, b_ref, o_ref):
    z = h_ref[...] + c_ref[...] + a_ref[...]
    o_ref[...] = jnp.dot(z, w_ref[...], preferred_element_type=jnp.float32) + b_ref[...]


_dec = pl.pallas_call(
    _dec_body,
    grid=(N // BMD,),
    in_specs=[
        pl.BlockSpec((BMD, D2), lambda i: (i, 0)),
        pl.BlockSpec((BMD, D2), lambda i: (i, 0)),
        pl.BlockSpec((BMD, D2), lambda i: (i, 0)),
        pl.BlockSpec((D2, OUT), lambda i: (0, 0)),
        pl.BlockSpec((1, OUT), lambda i: (0, 0)),
    ],
    out_specs=pl.BlockSpec((BMD, OUT), lambda i: (i, 0)),
    out_shape=jax.ShapeDtypeStruct((N, OUT), jnp.float32),
)

_layer_mid = _make_layer(True)
_layer_last = _make_layer(False)


def kernel(x, edge_index, edge_attr, masked_atom_mask, enc_W1, enc_b1,
           enc_W2, enc_b2, prelu_a, W_n2d, dec_W, dec_b):
    src = edge_index[0]
    dst = edge_index[1]
    x_pad = jnp.pad(x, ((0, TN - N), (0, D2 - D)))
    ea_pad = jnp.pad(edge_attr, ((0, TE - E), (0, D2 - D)))
    w1p = jnp.pad(enc_W1, ((0, 0), (0, D2 - D), (0, D2 - D)))
    b1p = jnp.pad(enc_b1, ((0, 0), (0, D2 - D)))
    w2p = jnp.pad(enc_W2, ((0, 0), (0, D2 - D), (0, D2 - D)))
    b2p = jnp.pad(enc_b2, ((0, 0), (0, D2 - D)))
    wn2dp = jnp.pad(W_n2d, ((0, D2 - D), (0, D2 - D)))
    dec_wp = jnp.pad(dec_W, ((0, D2 - D), (0, 0)))
    keep = jnp.pad(1.0 - masked_atom_mask.astype(jnp.float32).reshape(N, 1),
                   ((0, TN - N), (0, 0)))
    zeros_stage = jnp.zeros((TR * D2,), jnp.float32)

    ssrc, seid = _strips(src, dst)

    C = _sc_pass(ea_pad, seid, zeros_stage).reshape(NPAD, D2)

    h = x_pad
    for l in range(L):
        agg = _sc_pass(h, ssrc, zeros_stage).reshape(NPAD, D2)
        layer = _layer_mid if l < L - 1 else _layer_last
        h = layer(h, C, agg, w1p[l], b1p[l].reshape(1, D2),
                  w2p[l], b2p[l].reshape(1, D2))

    h = _n2d(h, keep, wn2dp, prelu_a.reshape(1, 1))

    agg = _sc_pass(h, ssrc, zeros_stage).reshape(NPAD, D2)
    return _dec(h, C, agg, dec_wp, dec_b.reshape(1, OUT))
